# baseline (speedup 1.0000x reference)
import jax
import jax.numpy as jnp
import numpy as np
from jax.experimental import pallas as pl
from jax.experimental.pallas import tpu as pltpu
from jax.sharding import Mesh, PartitionSpec as P
from jax.experimental.shard_map import shard_map

_N = 10000
_D = 512
_BM = 200


def _gcn_block_kernel(adj_ref, emb_ref, out_ref, emb_bf16):
    @pl.when(pl.program_id(0) == 0)
    def _cast_embeds_once():
        emb_bf16[...] = emb_ref[...].astype(jnp.bfloat16)

    a = adj_ref[...].astype(jnp.bfloat16)
    out_ref[...] = jnp.dot(a, emb_bf16[...], preferred_element_type=jnp.float32)


def _local_spmm(adj_shard, embeds):
    rows = adj_shard.shape[0]
    return pl.pallas_call(
        _gcn_block_kernel,
        grid=(rows // _BM,),
        in_specs=[
            pl.BlockSpec((_BM, _N), lambda i: (i, 0)),
            pl.BlockSpec((_N, _D), lambda i: (0, 0)),
        ],
        out_specs=pl.BlockSpec((_BM, _D), lambda i: (i, 0)),
        out_shape=jax.ShapeDtypeStruct((rows, _D), jnp.float32),
        scratch_shapes=[pltpu.VMEM((_N, _D), jnp.bfloat16)],
    )(adj_shard, embeds)


def kernel(adj, embeds):
    devs = jax.devices()
    nd = len(devs)
    if nd < 2 or _N % (nd * _BM) != 0:
        return _local_spmm(adj, embeds)
    mesh = Mesh(np.array(devs), ("x",))
    f = shard_map(
        _local_spmm,
        mesh=mesh,
        in_specs=(P("x", None), P(None, None)),
        out_specs=P("x", None),
        check_rep=False,
    )
    return f(adj, embeds)


# speedup vs baseline: 4.2733x; 4.2733x over previous
import jax
import jax.numpy as jnp
from jax.experimental import pallas as pl
from jax.experimental.pallas import tpu as pltpu

_N = 10000
_D = 512
_BM = 200


def _gcn_block_kernel(adj_ref, emb_ref, out_ref, emb_bf16):
    @pl.when(pl.program_id(0) == 0)
    def _cast_embeds_once():
        emb_bf16[...] = emb_ref[...].astype(jnp.bfloat16)

    a = adj_ref[...].astype(jnp.bfloat16)
    out_ref[...] = jnp.dot(a, emb_bf16[...], preferred_element_type=jnp.float32)


def _local_spmm(adj_shard, embeds):
    rows = adj_shard.shape[0]
    return pl.pallas_call(
        _gcn_block_kernel,
        grid=(rows // _BM,),
        in_specs=[
            pl.BlockSpec((_BM, _N), lambda i: (i, 0)),
            pl.BlockSpec((_N, _D), lambda i: (0, 0)),
        ],
        out_specs=pl.BlockSpec((_BM, _D), lambda i: (i, 0)),
        out_shape=jax.ShapeDtypeStruct((rows, _D), jnp.float32),
        scratch_shapes=[pltpu.VMEM((_N, _D), jnp.bfloat16)],
    )(adj_shard, embeds)


def kernel(adj, embeds):
    return _local_spmm(adj, embeds)


# speedup vs baseline: 4.7471x; 1.1109x over previous
import jax
import jax.numpy as jnp
from jax.experimental import pallas as pl
from jax.experimental.pallas import tpu as pltpu

_N = 10000
_D = 512
_BM = 400


def _gcn_block_kernel(adj_ref, emb_ref, out_ref):
    a = adj_ref[...].astype(jnp.bfloat16)
    e = emb_ref[...].astype(jnp.bfloat16)
    out_ref[...] = jnp.dot(a, e, preferred_element_type=jnp.float32)


def _local_spmm(adj_shard, embeds):
    rows = adj_shard.shape[0]
    return pl.pallas_call(
        _gcn_block_kernel,
        grid=(rows // _BM,),
        in_specs=[
            pl.BlockSpec((_BM, _N), lambda i: (i, 0)),
            pl.BlockSpec((_N, _D), lambda i: (0, 0)),
        ],
        out_specs=pl.BlockSpec((_BM, _D), lambda i: (i, 0)),
        out_shape=jax.ShapeDtypeStruct((rows, _D), jnp.float32),
    )(adj_shard, embeds)


def kernel(adj, embeds):
    return _local_spmm(adj, embeds)
